# Initial kernel scaffold; baseline (speedup 1.0000x reference)
import jax
import jax.numpy as jnp
from jax.experimental import pallas as pl


def kernel(x, edge_index, W1l, b1l, W1r, W2l, b2l, W2r, Wout, bout):
    raise NotImplementedError("write your pallas kernel here")


# speedup vs baseline: 4.1267x; 4.1267x over previous
import jax
import jax.numpy as jnp
from jax import lax
from jax.experimental import pallas as pl
from jax.experimental.pallas import tpu as pltpu
from jax.experimental.pallas import tpu_sc as plsc

N = 10000
N_PAD = 10240
E = 320000
D = 64
NC = 2
NS = 16
LANES = 16
NW = NC * NS
EDGES_PER_W = E // NW
K = 80
CHUNKS = EDGES_PER_W // K
RPT = N_PAD // NS
ZR = 128
DEGW = 16


def _sc_segsum(num_tables, with_deg):
  mesh = plsc.VectorSubcoreMesh(
      core_axis_name="c", subcore_axis_name="s",
      num_cores=NC, num_subcores=NS)
  out_types = [jax.ShapeDtypeStruct((NC * N_PAD, D), jnp.float32)
               for _ in range(num_tables)]
  scratch = [
      pltpu.VMEM((K,), jnp.int32),
      pltpu.VMEM((K,), jnp.int32),
      pltpu.VMEM((K, D), jnp.float32),
      pltpu.VMEM((ZR, D), jnp.float32),
      pltpu.VMEM((ZR, D), jnp.float32),
      pltpu.VMEM_SHARED((N_PAD, D), jnp.float32),
      pltpu.SemaphoreType.DMA,
  ]
  if with_deg:
    out_types.append(jax.ShapeDtypeStruct((NC * N_PAD, DEGW), jnp.float32))
    scratch += [
        pltpu.VMEM((K, DEGW), jnp.float32),
        pltpu.VMEM((RPT, DEGW), jnp.float32),
        pltpu.VMEM_SHARED((N_PAD, DEGW), jnp.float32),
    ]

  def body(*refs):
    tables = refs[:num_tables]
    srcs, dsts = refs[num_tables:num_tables + 2]
    rest = refs[num_tables + 2:]
    outs = rest[:num_tables]
    rest = rest[num_tables:]
    if with_deg:
      dout, srcv, dstv, rows, zbuf, obuf, acc, sem, ones, dbuf, dacc = rest
    else:
      srcv, dstv, rows, zbuf, obuf, acc, sem = rest
    cid = lax.axis_index("c")
    sid = lax.axis_index("s")
    wid = cid * NS + sid
    rbase = sid * RPT
    obase = cid * N_PAD + rbase
    ebase = wid * EDGES_PER_W

    def zrow(i, _):
      for j in range(D // LANES):
        zbuf[i, pl.ds(j * LANES, LANES)] = jnp.zeros((LANES,), jnp.float32)
      return 0
    lax.fori_loop(0, ZR, zrow, 0)
    if with_deg:
      def zdrow(i, _):
        dbuf[i, :] = jnp.zeros((DEGW,), jnp.float32)
        return 0
      lax.fori_loop(0, RPT, zdrow, 0)
      pltpu.sync_copy(dbuf, dacc.at[pl.ds(rbase, RPT)])
      def orow(i, _):
        ones[i, :] = jnp.ones((DEGW,), jnp.float32)
        return 0
      lax.fori_loop(0, K, orow, 0)

    for t in range(num_tables):
      for i in range(RPT // ZR):
        pltpu.sync_copy(zbuf, acc.at[pl.ds(rbase + i * ZR, ZR)])
      plsc.subcore_barrier()

      first = with_deg and t == 0
      def chunk(c, _):
        off = ebase + c * K
        pltpu.sync_copy(srcs.at[pl.ds(off, K)], srcv)
        pltpu.sync_copy(dsts.at[pl.ds(off, K)], dstv)
        pltpu.async_copy(tables[t].at[srcv], rows, sem).wait()
        pltpu.sync_copy(rows, acc.at[dstv], add=True)
        if first:
          pltpu.sync_copy(ones, dacc.at[dstv], add=True)
        return 0
      lax.fori_loop(0, CHUNKS, chunk, 0)
      plsc.subcore_barrier()

      for i in range(RPT // ZR):
        pltpu.sync_copy(acc.at[pl.ds(rbase + i * ZR, ZR)], obuf)
        pltpu.sync_copy(obuf, outs[t].at[pl.ds(obase + i * ZR, ZR)])
      plsc.subcore_barrier()

    if with_deg:
      pltpu.sync_copy(dacc.at[pl.ds(rbase, RPT)], dbuf)
      pltpu.sync_copy(dbuf, dout.at[pl.ds(obase, RPT)])

  out_type = tuple(out_types) if len(out_types) > 1 else out_types[0]
  return pl.kernel(
      body, out_type=out_type, mesh=mesh, scratch_types=scratch,
      compiler_params=pltpu.CompilerParams(use_tc_tiling_on_sc=False))


_RB = 400


def _mmt(a, w):
  return lax.dot_general(a, w, (((1,), (1,)), ((), ())),
                         preferred_element_type=jnp.float32)


def _tc_pre(x, W1l, W1r, b1l):
  def body(x_ref, wl_ref, wr_ref, b_ref, xla_ref, xlb_ref, xr_ref):
    xb = x_ref[...]
    xl = _mmt(xb, wl_ref[...])
    xla_ref[...] = xl[:, :64]
    xlb_ref[...] = xl[:, 64:]
    xr_ref[...] = _mmt(xb, wr_ref[...]) + b_ref[...]
  return pl.pallas_call(
      body,
      grid=(N // _RB,),
      in_specs=[
          pl.BlockSpec((_RB, 128), lambda i: (i, 0)),
          pl.BlockSpec((128, 128), lambda i: (0, 0)),
          pl.BlockSpec((128, 128), lambda i: (0, 0)),
          pl.BlockSpec((1, 128), lambda i: (0, 0)),
      ],
      out_specs=[pl.BlockSpec((_RB, 64), lambda i: (i, 0)),
                 pl.BlockSpec((_RB, 64), lambda i: (i, 0)),
                 pl.BlockSpec((_RB, 128), lambda i: (i, 0))],
      out_shape=[jax.ShapeDtypeStruct((N, 64), jnp.float32),
                 jax.ShapeDtypeStruct((N, 64), jnp.float32),
                 jax.ShapeDtypeStruct((N, 128), jnp.float32)],
  )(x, W1l, W1r, b1l.reshape(1, 128))


def _tc_mid(aa0, aa1, ab0, ab1, da, db, xr, W2l, W2r, b2l):
  def body(aa0_ref, aa1_ref, ab0_ref, ab1_ref, da_ref, db_ref, xr_ref,
           wl_ref, wr_ref, b_ref, hl_ref, hr_ref):
    deg = jnp.maximum((da_ref[...] + db_ref[...])[:, 0:1], 1.0)
    xr = xr_ref[...]
    h1a = jnp.maximum((aa0_ref[...] + aa1_ref[...]) / deg + xr[:, :64], 0.0)
    h1b = jnp.maximum((ab0_ref[...] + ab1_ref[...]) / deg + xr[:, 64:], 0.0)
    wl = wl_ref[...]
    wr = wr_ref[...]
    hl_ref[...] = _mmt(h1a, wl[:, :64]) + _mmt(h1b, wl[:, 64:])
    hr_ref[...] = _mmt(h1a, wr[:, :64]) + _mmt(h1b, wr[:, 64:]) + b_ref[...]
  spec64 = pl.BlockSpec((_RB, 64), lambda i: (i, 0))
  return pl.pallas_call(
      body,
      grid=(N // _RB,),
      in_specs=[
          spec64, spec64, spec64, spec64,
          pl.BlockSpec((_RB, DEGW), lambda i: (i, 0)),
          pl.BlockSpec((_RB, DEGW), lambda i: (i, 0)),
          pl.BlockSpec((_RB, 128), lambda i: (i, 0)),
          pl.BlockSpec((64, 128), lambda i: (0, 0)),
          pl.BlockSpec((64, 128), lambda i: (0, 0)),
          pl.BlockSpec((1, 64), lambda i: (0, 0)),
      ],
      out_specs=[spec64, spec64],
      out_shape=[jax.ShapeDtypeStruct((N, 64), jnp.float32)] * 2,
  )(aa0, aa1, ab0, ab1, da, db, xr, W2l, W2r, b2l.reshape(1, 64))


def _tc_post(aa, ab, da, db, hr, wo_pad, bo_pad):
  def body(aa_ref, ab_ref, da_ref, db_ref, hr_ref, wo_ref, bo_ref, out_ref):
    deg = jnp.maximum((da_ref[...] + db_ref[...])[:, 0:1], 1.0)
    h2 = jnp.maximum((aa_ref[...] + ab_ref[...]) / deg + hr_ref[...], 0.0)
    logits = _mmt(h2, wo_ref[...]) + bo_ref[...]
    m = jnp.max(logits, axis=1, keepdims=True)
    e = jnp.exp(logits - m)
    out_ref[...] = e / jnp.sum(e, axis=1, keepdims=True)
  spec64 = pl.BlockSpec((_RB, 64), lambda i: (i, 0))
  return pl.pallas_call(
      body,
      grid=(N // _RB,),
      in_specs=[
          spec64, spec64,
          pl.BlockSpec((_RB, DEGW), lambda i: (i, 0)),
          pl.BlockSpec((_RB, DEGW), lambda i: (i, 0)),
          spec64,
          pl.BlockSpec((64, 64), lambda i: (0, 0)),
          pl.BlockSpec((1, 64), lambda i: (0, 0)),
      ],
      out_specs=spec64,
      out_shape=jax.ShapeDtypeStruct((N, 64), jnp.float32),
  )(aa, ab, da, db, hr, wo_pad, bo_pad)


@jax.jit
def kernel(x, edge_index, W1l, b1l, W1r, W2l, b2l, W2r, Wout, bout):
  src = edge_index[0]
  dst = edge_index[1]

  xla, xlb, xr1 = _tc_pre(x, W1l, W1r, b1l)
  agg1a, agg1b, deg = _sc_segsum(2, True)(xla, xlb, src, dst)
  hl2, hr2 = _tc_mid(agg1a[:N], agg1a[N_PAD:N_PAD + N],
                     agg1b[:N], agg1b[N_PAD:N_PAD + N],
                     deg[:N], deg[N_PAD:N_PAD + N], xr1, W2l, W2r, b2l)

  agg2 = _sc_segsum(1, False)(hl2, src, dst)
  wo_pad = jnp.pad(Wout, ((0, 64 - Wout.shape[0]), (0, 0)))
  bo_pad = jnp.pad(bout, (0, 64 - bout.shape[0]), constant_values=-1e30)
  out = _tc_post(agg2[:N], agg2[N_PAD:N_PAD + N], deg[:N],
                 deg[N_PAD:N_PAD + N], hr2, wo_pad, bo_pad.reshape(1, 64))
  return out[:, :40]


# speedup vs baseline: 9.9915x; 2.4212x over previous
import jax
import jax.numpy as jnp
from jax import lax
from jax.experimental import pallas as pl
from jax.experimental.pallas import tpu as pltpu
from jax.experimental.pallas import tpu_sc as plsc

N = 10000
N_PAD = 10240
E = 320000
D = 64
NC = 2
NS = 16
LANES = 16
NW = NC * NS
K = 125
CHUNKS = E // (NW * K)
HALF = CHUNKS // 2
RPT = N_PAD // NS
ZR = 128
DEGW = 16


def _sc_segsum(num_tables, with_deg):
  mesh = plsc.VectorSubcoreMesh(
      core_axis_name="c", subcore_axis_name="s",
      num_cores=NC, num_subcores=NS)
  out_types = [jax.ShapeDtypeStruct((NC * N_PAD, D), jnp.float32)
               for _ in range(num_tables)]
  scratch = [
      pltpu.VMEM((CHUNKS, K), jnp.int32),
      pltpu.VMEM((CHUNKS, K), jnp.int32),
      pltpu.VMEM((K, D), jnp.float32),
      pltpu.VMEM((K, D), jnp.float32),
      pltpu.VMEM((ZR, D), jnp.float32),
      pltpu.VMEM((ZR, D), jnp.float32),
      pltpu.VMEM_SHARED((N_PAD, D), jnp.float32),
      pltpu.SemaphoreType.DMA,
      pltpu.SemaphoreType.DMA,
  ]
  if with_deg:
    out_types.append(jax.ShapeDtypeStruct((NC * N_PAD, DEGW), jnp.float32))
    scratch += [
        pltpu.VMEM((K, DEGW), jnp.float32),
        pltpu.VMEM((RPT, DEGW), jnp.float32),
        pltpu.VMEM_SHARED((N_PAD, DEGW), jnp.float32),
    ]

  def body(*refs):
    tables = refs[:num_tables]
    srcs, dsts = refs[num_tables:num_tables + 2]
    rest = refs[num_tables + 2:]
    outs = rest[:num_tables]
    rest = rest[num_tables:]
    if with_deg:
      (dout, srcv, dstv, rows0, rows1, zbuf, obuf, acc, sem0, sem1,
       ones, dbuf, dacc) = rest
    else:
      srcv, dstv, rows0, rows1, zbuf, obuf, acc, sem0, sem1 = rest
    rows = (rows0, rows1)
    sems = (sem0, sem1)
    cid = lax.axis_index("c")
    sid = lax.axis_index("s")
    wid = cid * NS + sid
    rbase = sid * RPT
    obase = cid * N_PAD + rbase

    pltpu.sync_copy(srcs.at[wid], srcv)
    pltpu.sync_copy(dsts.at[wid], dstv)

    def zrow(i, _):
      for j in range(D // LANES):
        zbuf[i, pl.ds(j * LANES, LANES)] = jnp.zeros((LANES,), jnp.float32)
      return 0
    lax.fori_loop(0, ZR, zrow, 0)
    if with_deg:
      def zdrow(i, _):
        dbuf[i, :] = jnp.zeros((DEGW,), jnp.float32)
        return 0
      lax.fori_loop(0, RPT, zdrow, 0)
      pltpu.sync_copy(dbuf, dacc.at[pl.ds(rbase, RPT)])
      def orow(i, _):
        ones[i, :] = jnp.ones((DEGW,), jnp.float32)
        return 0
      lax.fori_loop(0, K, orow, 0)

    for t in range(num_tables):
      for i in range(RPT // ZR):
        pltpu.sync_copy(zbuf, acc.at[pl.ds(rbase + i * ZR, ZR)])
      plsc.subcore_barrier()

      first = with_deg and t == 0
      table = tables[t]

      def fire(c, b):
        pltpu.async_copy(table.at[srcv.at[c]], rows[b], sems[b])

      def drain(b):
        pltpu.make_async_copy(table.at[srcv.at[0]], rows[b], sems[b]).wait()

      def scat(c, b):
        pltpu.sync_copy(rows[b], acc.at[dstv.at[c]], add=True)
        if first:
          pltpu.sync_copy(ones, dacc.at[dstv.at[c]], add=True)

      fire(0, 0)
      fire(1, 1)
      def step(i, _):
        for b in range(2):
          c = 2 * i + b
          drain(b)
          scat(c, b)
          fire(c + 2, b)
        return 0
      lax.fori_loop(0, HALF - 1, step, 0)
      for b in range(2):
        c = CHUNKS - 2 + b
        drain(b)
        scat(c, b)
      plsc.subcore_barrier()

      for i in range(RPT // ZR):
        pltpu.sync_copy(acc.at[pl.ds(rbase + i * ZR, ZR)], obuf)
        pltpu.sync_copy(obuf, outs[t].at[pl.ds(obase + i * ZR, ZR)])
      plsc.subcore_barrier()

    if with_deg:
      pltpu.sync_copy(dacc.at[pl.ds(rbase, RPT)], dbuf)
      pltpu.sync_copy(dbuf, dout.at[pl.ds(obase, RPT)])

  out_type = tuple(out_types) if len(out_types) > 1 else out_types[0]
  return pl.kernel(
      body, out_type=out_type, mesh=mesh, scratch_types=scratch,
      compiler_params=pltpu.CompilerParams(use_tc_tiling_on_sc=False))


_RB = 400


def _mmt(a, w):
  return lax.dot_general(a, w, (((1,), (1,)), ((), ())),
                         preferred_element_type=jnp.float32)


def _tc_pre(x, W1l, W1r, b1l):
  def body(x_ref, wl_ref, wr_ref, b_ref, xla_ref, xlb_ref, xr_ref):
    xb = x_ref[...]
    xl = _mmt(xb, wl_ref[...])
    xla_ref[...] = xl[:, :64]
    xlb_ref[...] = xl[:, 64:]
    xr_ref[...] = _mmt(xb, wr_ref[...]) + b_ref[...]
  return pl.pallas_call(
      body,
      grid=(N // _RB,),
      in_specs=[
          pl.BlockSpec((_RB, 128), lambda i: (i, 0)),
          pl.BlockSpec((128, 128), lambda i: (0, 0)),
          pl.BlockSpec((128, 128), lambda i: (0, 0)),
          pl.BlockSpec((1, 128), lambda i: (0, 0)),
      ],
      out_specs=[pl.BlockSpec((_RB, 64), lambda i: (i, 0)),
                 pl.BlockSpec((_RB, 64), lambda i: (i, 0)),
                 pl.BlockSpec((_RB, 128), lambda i: (i, 0))],
      out_shape=[jax.ShapeDtypeStruct((N, 64), jnp.float32),
                 jax.ShapeDtypeStruct((N, 64), jnp.float32),
                 jax.ShapeDtypeStruct((N, 128), jnp.float32)],
  )(x, W1l, W1r, b1l.reshape(1, 128))


def _tc_mid(aa0, aa1, ab0, ab1, da, db, xr, W2l, W2r, b2l):
  def body(aa0_ref, aa1_ref, ab0_ref, ab1_ref, da_ref, db_ref, xr_ref,
           wl_ref, wr_ref, b_ref, hl_ref, hr_ref):
    deg = jnp.maximum((da_ref[...] + db_ref[...])[:, 0:1], 1.0)
    xr = xr_ref[...]
    h1a = jnp.maximum((aa0_ref[...] + aa1_ref[...]) / deg + xr[:, :64], 0.0)
    h1b = jnp.maximum((ab0_ref[...] + ab1_ref[...]) / deg + xr[:, 64:], 0.0)
    wl = wl_ref[...]
    wr = wr_ref[...]
    hl_ref[...] = _mmt(h1a, wl[:, :64]) + _mmt(h1b, wl[:, 64:])
    hr_ref[...] = _mmt(h1a, wr[:, :64]) + _mmt(h1b, wr[:, 64:]) + b_ref[...]
  spec64 = pl.BlockSpec((_RB, 64), lambda i: (i, 0))
  return pl.pallas_call(
      body,
      grid=(N // _RB,),
      in_specs=[
          spec64, spec64, spec64, spec64,
          pl.BlockSpec((_RB, DEGW), lambda i: (i, 0)),
          pl.BlockSpec((_RB, DEGW), lambda i: (i, 0)),
          pl.BlockSpec((_RB, 128), lambda i: (i, 0)),
          pl.BlockSpec((64, 128), lambda i: (0, 0)),
          pl.BlockSpec((64, 128), lambda i: (0, 0)),
          pl.BlockSpec((1, 64), lambda i: (0, 0)),
      ],
      out_specs=[spec64, spec64],
      out_shape=[jax.ShapeDtypeStruct((N, 64), jnp.float32)] * 2,
  )(aa0, aa1, ab0, ab1, da, db, xr, W2l, W2r, b2l.reshape(1, 64))


def _tc_post(aa, ab, da, db, hr, wo_pad, bo_pad):
  def body(aa_ref, ab_ref, da_ref, db_ref, hr_ref, wo_ref, bo_ref, out_ref):
    deg = jnp.maximum((da_ref[...] + db_ref[...])[:, 0:1], 1.0)
    h2 = jnp.maximum((aa_ref[...] + ab_ref[...]) / deg + hr_ref[...], 0.0)
    logits = _mmt(h2, wo_ref[...]) + bo_ref[...]
    m = jnp.max(logits, axis=1, keepdims=True)
    e = jnp.exp(logits - m)
    out_ref[...] = e / jnp.sum(e, axis=1, keepdims=True)
  spec64 = pl.BlockSpec((_RB, 64), lambda i: (i, 0))
  return pl.pallas_call(
      body,
      grid=(N // _RB,),
      in_specs=[
          spec64, spec64,
          pl.BlockSpec((_RB, DEGW), lambda i: (i, 0)),
          pl.BlockSpec((_RB, DEGW), lambda i: (i, 0)),
          spec64,
          pl.BlockSpec((64, 64), lambda i: (0, 0)),
          pl.BlockSpec((1, 64), lambda i: (0, 0)),
      ],
      out_specs=spec64,
      out_shape=jax.ShapeDtypeStruct((N, 64), jnp.float32),
  )(aa, ab, da, db, hr, wo_pad, bo_pad)


@jax.jit
def kernel(x, edge_index, W1l, b1l, W1r, W2l, b2l, W2r, Wout, bout):
  src = edge_index[0].reshape(NW, CHUNKS, K)
  dst = edge_index[1].reshape(NW, CHUNKS, K)

  xla, xlb, xr1 = _tc_pre(x, W1l, W1r, b1l)
  agg1a, agg1b, deg = _sc_segsum(2, True)(xla, xlb, src, dst)
  hl2, hr2 = _tc_mid(agg1a[:N], agg1a[N_PAD:N_PAD + N],
                     agg1b[:N], agg1b[N_PAD:N_PAD + N],
                     deg[:N], deg[N_PAD:N_PAD + N], xr1, W2l, W2r, b2l)

  agg2 = _sc_segsum(1, False)(hl2, src, dst)
  wo_pad = jnp.pad(Wout, ((0, 64 - Wout.shape[0]), (0, 0)))
  bo_pad = jnp.pad(bout, (0, 64 - bout.shape[0]), constant_values=-1e30)
  out = _tc_post(agg2[:N], agg2[N_PAD:N_PAD + N], deg[:N],
                 deg[N_PAD:N_PAD + N], hr2, wo_pad, bo_pad.reshape(1, 64))
  return out[:, :40]
